# baseline (speedup 1.0000x reference)
import dataclasses
import functools

import jax
import jax.numpy as jnp
from jax import lax
from jax.experimental import pallas as pl
from jax.experimental.pallas import tpu as pltpu
from jax.experimental.pallas import tpu_sc as plsc

_N, _E, _DN, _DE, _H, _NG = 10000, 320000, 128, 16, 128, 16

_NBLK = 1000
_EBLK = 1280
_GCH = 64
_NGCHUNK = _E // _GCH
_NW = 32
_NPAD = 10240
_RPT = _NPAD // 16


def _rne_bf16_bits(x):
    b = lax.bitcast_convert_type(x, jnp.int32)
    return b + 0x7FFF + (lax.shift_right_logical(b, 16) & 1)


def _pack2bf16(f):
    h = f.shape[1] // 2
    lo = lax.shift_right_logical(_rne_bf16_bits(f[:, :h]), 16)
    hi = _rne_bf16_bits(f[:, h:]) & jnp.int32(-65536)
    return lo | hi


def _unpack2bf16(w):
    lo = lax.bitcast_convert_type(lax.shift_left(w, 16), jnp.float32)
    hi = lax.bitcast_convert_type(w & jnp.int32(-65536), jnp.float32)
    return lo, hi


def _ln_rows(y, g, b):
    m = jnp.mean(y, axis=-1, keepdims=True)
    v = jnp.mean((y - m) ** 2, axis=-1, keepdims=True)
    return (y - m) / jnp.sqrt(v + 1e-5) * g + b


def _dot16(a, b):
    return jnp.dot(a.astype(jnp.bfloat16), b.astype(jnp.bfloat16),
                   preferred_element_type=jnp.float32)


def _mlp_ln_body(x_ref, w1, b1, w2, b2, g, be, o_ref, *, pack=False):
    h = jnp.maximum(_dot16(x_ref[...], w1[...]) + b1[...], 0.0)
    y = _dot16(h, w2[...]) + b2[...]
    y = _ln_rows(y, g[...], be[...])
    if pack:
        o_ref[...] = _pack2bf16(y)
    else:
        o_ref[...] = y


def _encoder(x, w1t, b1, w2t, b2, g, be, blk, grid, pack=False):
    din = x.shape[1]
    hout = _H // 2 if pack else _H
    odt = jnp.int32 if pack else jnp.float32
    return pl.pallas_call(
        functools.partial(_mlp_ln_body, pack=pack),
        grid=(grid,),
        in_specs=[
            pl.BlockSpec((blk, din), lambda i: (i, 0)),
            pl.BlockSpec((din, _H), lambda i: (0, 0)),
            pl.BlockSpec((1, _H), lambda i: (0, 0)),
            pl.BlockSpec((_H, _H), lambda i: (0, 0)),
            pl.BlockSpec((1, _H), lambda i: (0, 0)),
            pl.BlockSpec((1, _H), lambda i: (0, 0)),
            pl.BlockSpec((1, _H), lambda i: (0, 0)),
        ],
        out_specs=pl.BlockSpec((blk, hout), lambda i: (i, 0)),
        out_shape=jax.ShapeDtypeStruct((x.shape[0], hout), odt),
    )(x, w1t, b1, w2t, b2, g, be)


def _node_mlp_body(eps_ref, x_ref, a_ref, w1, b1, w2, b2, g, be, o_ref,
                   *, post_relu):
    z = eps_ref[0, 0] * x_ref[...] + a_ref[0] + a_ref[1]
    h = jnp.maximum(_dot16(z, w1[...]) + b1[...], 0.0)
    y = _dot16(h, w2[...]) + b2[...]
    y = _ln_rows(y, g[...], be[...])
    if post_relu:
        y = jnp.maximum(y, 0.0)
    o_ref[...] = y


def _node_mlp(eps1, x, aggr2, w1t, b1, w2t, b2, g, be, post_relu):
    return pl.pallas_call(
        functools.partial(_node_mlp_body, post_relu=post_relu),
        grid=(_N // _NBLK,),
        in_specs=[
            pl.BlockSpec(memory_space=pltpu.SMEM),
            pl.BlockSpec((_NBLK, _H), lambda i: (i, 0)),
            pl.BlockSpec((2, _NBLK, _H), lambda i: (0, i, 0)),
            pl.BlockSpec((_H, _H), lambda i: (0, 0)),
            pl.BlockSpec((1, _H), lambda i: (0, 0)),
            pl.BlockSpec((_H, _H), lambda i: (0, 0)),
            pl.BlockSpec((1, _H), lambda i: (0, 0)),
            pl.BlockSpec((1, _H), lambda i: (0, 0)),
            pl.BlockSpec((1, _H), lambda i: (0, 0)),
        ],
        out_specs=pl.BlockSpec((_NBLK, _H), lambda i: (i, 0)),
        out_shape=jax.ShapeDtypeStruct((_N, _H), jnp.float32),
    )(eps1, x, aggr2, w1t, b1, w2t, b2, g, be)


def _pool_sums_body(x_ref, b_ref, sums_ref, cnt_ref):
    i = pl.program_id(0)
    bb = b_ref[...]
    iota = lax.broadcasted_iota(jnp.int32, (_NBLK, _NG), 1)
    onehot = (bb == iota).astype(jnp.float32)
    dn = (((0,), (0,)), ((), ()))
    s = lax.dot_general(onehot, x_ref[...], dn,
                        preferred_element_type=jnp.float32)
    c = lax.dot_general(onehot, jnp.ones_like(x_ref[...]), dn,
                        preferred_element_type=jnp.float32)

    @pl.when(i == 0)
    def _():
        sums_ref[...] = s
        cnt_ref[...] = c

    @pl.when(i > 0)
    def _():
        sums_ref[...] += s
        cnt_ref[...] += c


def _pool_sums(x2, batch2d):
    return pl.pallas_call(
        _pool_sums_body,
        grid=(_N // _NBLK,),
        in_specs=[
            pl.BlockSpec((_NBLK, _H), lambda i: (i, 0)),
            pl.BlockSpec((_NBLK, 1), lambda i: (i, 0)),
        ],
        out_specs=[
            pl.BlockSpec((_NG, _H), lambda i: (0, 0)),
            pl.BlockSpec((_NG, _H), lambda i: (0, 0)),
        ],
        out_shape=[
            jax.ShapeDtypeStruct((_NG, _H), jnp.float32),
            jax.ShapeDtypeStruct((_NG, _H), jnp.float32),
        ],
    )(x2, batch2d)


def _graph_mlp_body(sums_ref, cnt_ref, wt, b, g, be, wct, gc_ref):
    gmean = sums_ref[...] / jnp.maximum(cnt_ref[...], 1.0)
    t = jnp.maximum(
        jnp.dot(gmean, wt[...], preferred_element_type=jnp.float32)
        + b[...], 0.0)
    gg = _ln_rows(t, g[...], be[...])
    gc_ref[...] = jnp.dot(gg, wct[...], preferred_element_type=jnp.float32)


def _graph_mlp(sums, cnt, wt, b, g, be, wct):
    return pl.pallas_call(
        _graph_mlp_body,
        out_shape=jax.ShapeDtypeStruct((_NG, 2 * _H), jnp.float32),
    )(sums, cnt, wt, b, g, be, wct)


def _proj_body(x_ref, b_ref, gc_ref, wat, wbt, p_ref, q_ref):
    bb = b_ref[...]
    iota = lax.broadcasted_iota(jnp.int32, (_NBLK, _NG), 1)
    onehot = (bb == iota).astype(jnp.float32)
    xb = x_ref[...]
    pf = (_dot16(xb, wat[...])
          + jnp.dot(onehot, gc_ref[...], preferred_element_type=jnp.float32))
    qf = _dot16(xb, wbt[...])
    p_ref[...] = _pack2bf16(pf)
    q_ref[...] = _pack2bf16(qf)


def _projections(x2, batch2d, gc, wat, wbt):
    return pl.pallas_call(
        _proj_body,
        grid=(_N // _NBLK,),
        in_specs=[
            pl.BlockSpec((_NBLK, _H), lambda i: (i, 0)),
            pl.BlockSpec((_NBLK, 1), lambda i: (i, 0)),
            pl.BlockSpec((_NG, 2 * _H), lambda i: (0, 0)),
            pl.BlockSpec((_H, 2 * _H), lambda i: (0, 0)),
            pl.BlockSpec((_H, 2 * _H), lambda i: (0, 0)),
        ],
        out_specs=[
            pl.BlockSpec((_NBLK, _H), lambda i: (i, 0)),
            pl.BlockSpec((_NBLK, _H), lambda i: (i, 0)),
        ],
        out_shape=[
            jax.ShapeDtypeStruct((_N, _H), jnp.int32),
            jax.ShapeDtypeStruct((_N, _H), jnp.int32),
        ],
    )(x2, batch2d, gc, wat, wbt)


def _final_body(e_ref, s_ref, wdlo, wdhi, b1lo, b1hi, w2lo, w2hi,
                b2, w3, b3, o_ref):
    ea, eb = _unpack2bf16(e_ref[...])
    ea = ea.astype(jnp.bfloat16)
    eb = eb.astype(jnp.bfloat16)
    hh = _H // 2
    telo = (jnp.dot(ea, wdlo[:hh, :], preferred_element_type=jnp.float32)
            + jnp.dot(eb, wdlo[hh:, :], preferred_element_type=jnp.float32))
    tehi = (jnp.dot(ea, wdhi[:hh, :], preferred_element_type=jnp.float32)
            + jnp.dot(eb, wdhi[hh:, :], preferred_element_type=jnp.float32))
    slo, shi = _unpack2bf16(s_ref[...])
    h1lo = jnp.tanh(slo + telo + b1lo[...])
    h1hi = jnp.tanh(shi + tehi + b1hi[...])
    h2 = jnp.tanh(
        jnp.dot(h1lo.astype(jnp.bfloat16), w2lo[...],
                preferred_element_type=jnp.float32)
        + jnp.dot(h1hi.astype(jnp.bfloat16), w2hi[...],
                  preferred_element_type=jnp.float32)
        + b2[...])
    o_ref[...] = jax.nn.sigmoid(
        jnp.sum(h2 * w3[...], axis=1, keepdims=True) + b3[...])


def _final_mlp(e_emb, s3, base, ne, *ws):
    wspec = pl.BlockSpec((_H, _H), lambda i: (0, 0))
    bspec = pl.BlockSpec((1, _H), lambda i: (0, 0))
    bblk = base // _EBLK
    return pl.pallas_call(
        _final_body,
        grid=(ne // _EBLK,),
        in_specs=[
            pl.BlockSpec((_EBLK, _H // 2), lambda i: (i + bblk, 0)),
            pl.BlockSpec((_EBLK, _H), lambda i: (i, 0)),
            wspec, wspec, bspec, bspec, wspec, wspec, bspec, bspec,
            pl.BlockSpec((1, 1), lambda i: (0, 0)),
        ],
        out_specs=pl.BlockSpec((_EBLK, 1), lambda i: (i, 0)),
        out_shape=jax.ShapeDtypeStruct((ne, 1), jnp.float32),
    )(e_emb, s3, *ws)


def _sc_mesh():
    return plsc.VectorSubcoreMesh(core_axis_name="c", subcore_axis_name="s")


def _sc_no_layout_params():
    cp = pltpu.CompilerParams()
    if "needs_layout_passes" in pltpu.CompilerParams.__dataclass_fields__:
        cp = dataclasses.replace(cp, needs_layout_passes=False)
    return cp


def _gine_aggregate(x_emb, e_emb, src, dst):
    K = _NGCHUNK // _NW
    NREM = _NGCHUNK - K * _NW

    @functools.partial(
        pl.kernel,
        out_type=jax.ShapeDtypeStruct((2, _NPAD, _H), jnp.float32),
        mesh=_sc_mesh(),
        compiler_params=_sc_no_layout_params(),
        scratch_types=[
            pltpu.VMEM((_GCH,), jnp.int32),
            pltpu.VMEM((_GCH,), jnp.int32),
            pltpu.VMEM((_GCH,), jnp.int32),
            pltpu.VMEM((_GCH,), jnp.int32),
            pltpu.VMEM((_GCH, _H), jnp.float32),
            pltpu.VMEM((_GCH, _H), jnp.float32),
            pltpu.VMEM((_GCH, _H // 2), jnp.int32),
            pltpu.VMEM((_GCH, _H // 2), jnp.int32),
            pltpu.VMEM_SHARED((_NPAD, _H), jnp.float32),
            pltpu.SemaphoreType.DMA,
            pltpu.SemaphoreType.DMA,
            pltpu.SemaphoreType.DMA,
            pltpu.SemaphoreType.DMA,
            pltpu.SemaphoreType.DMA,
            pltpu.SemaphoreType.DMA,
            pltpu.SemaphoreType.DMA,
        ],
    )
    def k(x_hbm, e_hbm, src_hbm, dst_hbm, out_hbm,
          sidx0, sidx1, didx0, didx1, xr0, xr1, er0, er1, accum,
          semi0, semi1, semg0, semg1, seme0, seme1, semw):
        semi, semg, seme = [semi0, semi1], [semg0, semg1], [seme0, seme1]
        sidx, didx, xr, er = [sidx0, sidx1], [didx0, didx1], [xr0, xr1], [er0, er1]
        cid = lax.axis_index("c")
        sid = lax.axis_index("s")
        wid = sid * 2 + cid

        @pl.loop(0, _GCH)
        def _(r):
            for c in range(_H // 16):
                xr0[r, pl.ds(c * 16, 16)] = jnp.zeros((16,), jnp.float32)

        for j in range(_RPT // _GCH):
            pltpu.sync_copy(xr0, accum.at[pl.ds(sid * _RPT + j * _GCH, _GCH)])
        plsc.subcore_barrier()

        def off(j):
            return (j * _NW + wid) * _GCH

        def issue_idx(j, b):
            pltpu.async_copy(src_hbm.at[pl.ds(off(j), _GCH)], sidx[b], semi[b])
            pltpu.async_copy(dst_hbm.at[pl.ds(off(j), _GCH)], didx[b], semi[b])

        def wait_idx(j, b):
            pltpu.make_async_copy(src_hbm.at[pl.ds(off(j), _GCH)], sidx[b],
                                  semi[b]).wait()
            pltpu.make_async_copy(dst_hbm.at[pl.ds(off(j), _GCH)], didx[b],
                                  semi[b]).wait()

        def issue_gather(j, b):
            pltpu.async_copy(x_hbm.at[sidx[b]], xr[b], semg[b])
            pltpu.async_copy(e_hbm.at[pl.ds(off(j), _GCH)], er[b], seme[b])

        def wait_gather(j, b):
            pltpu.make_async_copy(x_hbm.at[sidx[b]], xr[b], semg[b]).wait()
            pltpu.make_async_copy(e_hbm.at[pl.ds(off(j), _GCH)], er[b],
                                  seme[b]).wait()

        def compute(b):
            @pl.loop(0, _GCH)
            def _(r):
                for c in range(_H // 32):
                    w = plsc.bitcast(er[b][r, pl.ds(c * 16, 16)],
                                     jnp.bfloat16)
                    elo, ehi = plsc.unpack(w, format=plsc.PackFormat.INTERLEAVED)
                    sl = pl.ds(c * 16, 16)
                    sh = pl.ds(_H // 2 + c * 16, 16)
                    xr[b][r, sl] = jnp.maximum(xr[b][r, sl] + elo, 0.0)
                    xr[b][r, sh] = jnp.maximum(xr[b][r, sh] + ehi, 0.0)

        issue_idx(0, 0)
        issue_idx(1, 1)
        wait_idx(0, 0)
        issue_gather(0, 0)

        @pl.loop(0, K // 2)
        def _(jj):
            for u in range(2):
                j = 2 * jj + u
                b, nb = u, 1 - u

                @pl.when(j + 1 < K)
                def _():
                    wait_idx(j + 1, nb)
                    issue_gather(j + 1, nb)

                wait_gather(j, b)
                compute(b)
                pltpu.async_copy(xr[b], accum.at[didx[b]], semw, add=True)
                pltpu.make_async_copy(xr[b], accum.at[didx[b]], semw).wait()

                @pl.when(j + 2 < K)
                def _():
                    issue_idx(j + 2, b)

        @pl.when(wid < NREM)
        def _():
            o = (K * _NW + wid) * _GCH
            pltpu.sync_copy(src_hbm.at[pl.ds(o, _GCH)], sidx0)
            pltpu.sync_copy(dst_hbm.at[pl.ds(o, _GCH)], didx0)
            pltpu.async_copy(x_hbm.at[sidx0], xr0, semg0).wait()
            pltpu.sync_copy(e_hbm.at[pl.ds(o, _GCH)], er0)

            @pl.loop(0, _GCH)
            def _(r):
                for c in range(_H // 32):
                    w = plsc.bitcast(er0[r, pl.ds(c * 16, 16)], jnp.bfloat16)
                    elo, ehi = plsc.unpack(w, format=plsc.PackFormat.INTERLEAVED)
                    sl = pl.ds(c * 16, 16)
                    sh = pl.ds(_H // 2 + c * 16, 16)
                    xr0[r, sl] = jnp.maximum(xr0[r, sl] + elo, 0.0)
                    xr0[r, sh] = jnp.maximum(xr0[r, sh] + ehi, 0.0)

            pltpu.sync_copy(xr0, accum.at[didx0], add=True)

        plsc.subcore_barrier()
        pltpu.sync_copy(accum.at[pl.ds(sid * _RPT, _RPT)],
                        out_hbm.at[cid, pl.ds(sid * _RPT, _RPT)])

    return k(x_emb, e_emb, src, dst)


_CCH = 128
_NCCHUNK = _E // _CCH


def _edge_combine(p, q, src, dst, ne):
    nchunk = ne // _CCH
    K = (nchunk // _NW) & ~1
    NREM = nchunk - K * _NW

    @functools.partial(
        pl.kernel,
        out_type=jax.ShapeDtypeStruct((ne, _H), jnp.int32),
        mesh=_sc_mesh(),
        compiler_params=_sc_no_layout_params(),
        scratch_types=[
            pltpu.VMEM((_CCH,), jnp.int32),
            pltpu.VMEM((_CCH,), jnp.int32),
            pltpu.VMEM((_CCH,), jnp.int32),
            pltpu.VMEM((_CCH,), jnp.int32),
            pltpu.VMEM((_CCH, _H), jnp.int32),
            pltpu.VMEM((_CCH, _H), jnp.int32),
            pltpu.VMEM((_CCH, _H), jnp.int32),
            pltpu.VMEM((_CCH, _H), jnp.int32),
            pltpu.SemaphoreType.DMA,
            pltpu.SemaphoreType.DMA,
            pltpu.SemaphoreType.DMA,
            pltpu.SemaphoreType.DMA,
            pltpu.SemaphoreType.DMA,
        ],
    )
    def k(p_hbm, q_hbm, src_hbm, dst_hbm, out_hbm,
          sidx0, sidx1, didx0, didx1, pr0, pr1, qr0, qr1,
          semi0, semi1, semg0, semg1, semw):
        semi, semg = [semi0, semi1], [semg0, semg1]
        sidx, didx, pr, qr = [sidx0, sidx1], [didx0, didx1], [pr0, pr1], [qr0, qr1]
        cid = lax.axis_index("c")
        sid = lax.axis_index("s")
        wid = sid * 2 + cid

        def off(j):
            return (j * _NW + wid) * _CCH

        def issue_idx(j, b):
            pltpu.async_copy(src_hbm.at[pl.ds(off(j), _CCH)], sidx[b], semi[b])
            pltpu.async_copy(dst_hbm.at[pl.ds(off(j), _CCH)], didx[b], semi[b])

        def wait_idx(j, b):
            pltpu.make_async_copy(src_hbm.at[pl.ds(off(j), _CCH)], sidx[b],
                                  semi[b]).wait()
            pltpu.make_async_copy(dst_hbm.at[pl.ds(off(j), _CCH)], didx[b],
                                  semi[b]).wait()

        def issue_gather(b):
            pltpu.async_copy(p_hbm.at[sidx[b]], pr[b], semg[b])
            pltpu.async_copy(q_hbm.at[didx[b]], qr[b], semg[b])

        def wait_gather(b):
            pltpu.make_async_copy(p_hbm.at[sidx[b]], pr[b], semg[b]).wait()
            pltpu.make_async_copy(q_hbm.at[didx[b]], qr[b], semg[b]).wait()

        def compute(b):
            @pl.loop(0, _CCH)
            def _(r):
                for c in range(_H // 16):
                    sl = pl.ds(c * 16, 16)
                    s = (plsc.bitcast(pr[b][r, sl], jnp.bfloat16)
                         + plsc.bitcast(qr[b][r, sl], jnp.bfloat16))
                    pr[b][r, sl] = plsc.bitcast(s, jnp.int32)

        issue_idx(0, 0)
        issue_idx(1, 1)
        wait_idx(0, 0)
        issue_gather(0)

        @pl.loop(0, K // 2)
        def _(jj):
            for u in range(2):
                j = 2 * jj + u
                b, nb = u, 1 - u

                @pl.when(j + 1 < K)
                def _():
                    wait_idx(j + 1, nb)
                    issue_gather(nb)

                wait_gather(b)
                compute(b)
                pltpu.async_copy(pr[b], out_hbm.at[pl.ds(off(j), _CCH)],
                                 semw)
                pltpu.make_async_copy(pr[b],
                                      out_hbm.at[pl.ds(off(j), _CCH)],
                                      semw).wait()

                @pl.when(j + 2 < K)
                def _():
                    issue_idx(j + 2, b)

        for t in range((NREM + _NW - 1) // _NW):
            @pl.when((K + t) * _NW + wid < nchunk)
            def _():
                o = ((K + t) * _NW + wid) * _CCH
                pltpu.sync_copy(src_hbm.at[pl.ds(o, _CCH)], sidx0)
                pltpu.sync_copy(dst_hbm.at[pl.ds(o, _CCH)], didx0)
                pltpu.async_copy(p_hbm.at[sidx0], pr0, semg0).wait()
                pltpu.async_copy(q_hbm.at[didx0], qr0, semg0).wait()

                @pl.loop(0, _CCH)
                def _(r):
                    for c in range(_H // 16):
                        sl = pl.ds(c * 16, 16)
                        s = (plsc.bitcast(pr0[r, sl], jnp.bfloat16)
                             + plsc.bitcast(qr0[r, sl], jnp.bfloat16))
                        pr0[r, sl] = plsc.bitcast(s, jnp.int32)

                pltpu.sync_copy(pr0, out_hbm.at[pl.ds(o, _CCH)])

    return k(p, q, src, dst)


def kernel(x, edge_index, edge_attr, batch, params):
    p = params
    src = edge_index[0]
    dst = edge_index[1]
    batch2d = batch.astype(jnp.int32).reshape(_N, 1)

    def row(v):
        return v.reshape(1, -1)

    x_emb = _encoder(x, p['ne_w1'].T, row(p['ne_b1']), p['ne_w2'].T,
                     row(p['ne_b2']), row(p['ne_g']), row(p['ne_be']),
                     _NBLK, _N // _NBLK)
    e_emb = _encoder(edge_attr, p['ee_w1'].T, row(p['ee_b1']), p['ee_w2'].T,
                     row(p['ee_b2']), row(p['ee_g']), row(p['ee_be']),
                     _EBLK, _E // _EBLK, pack=True)

    x_curr = x_emb
    for i in range(2):
        aggr2 = _gine_aggregate(x_curr, e_emb, src, dst)
        eps1 = (1.0 + p['gin%d_eps' % i]).astype(jnp.float32).reshape(1, 1)
        x_curr = _node_mlp(
            eps1, x_curr, aggr2,
            p['gin%d_w1' % i].T, row(p['gin%d_b1' % i]),
            p['gin%d_w2' % i].T, row(p['gin%d_b2' % i]),
            row(p['gin%d_g' % i]), row(p['gin%d_be' % i]),
            post_relu=(i < 1))

    sums, cnt = _pool_sums(x_curr, batch2d)
    wc_t = p['ep_w1'][:, 2 * _H:3 * _H].T
    gc = _graph_mlp(sums, cnt, p['gp_w'].T, row(p['gp_b']), row(p['gp_g']),
                    row(p['gp_be']), wc_t)

    wa_t = p['ep_w1'][:, :_H].T
    wb_t = p['ep_w1'][:, _H:2 * _H].T
    pn, qn = _projections(x_curr, batch2d, gc, wa_t, wb_t)

    wd_t = p['ep_w1'][:, 3 * _H:].T.astype(jnp.bfloat16)
    w2_t = p['ep_w2'].T.astype(jnp.bfloat16)
    ws = (wd_t[:, :_H], wd_t[:, _H:],
          row(p['ep_b1'][:_H]), row(p['ep_b1'][_H:]),
          w2_t[:_H, :], w2_t[_H:, :],
          row(p['ep_b2']), row(p['ep_w3']),
          p['ep_b3'].reshape(1, 1))
    ea = 163840
    outs = []
    for base, ne in ((0, ea), (ea, _E - ea)):
        s3 = _edge_combine(pn, qn, src[base:base + ne], dst[base:base + ne],
                           ne)
        outs.append(_final_mlp(e_emb, s3, base, ne, *ws))
    return jnp.concatenate(outs, axis=0)


# speedup vs baseline: 1.0336x; 1.0336x over previous
import dataclasses
import functools

import jax
import jax.numpy as jnp
from jax import lax
from jax.experimental import pallas as pl
from jax.experimental.pallas import tpu as pltpu
from jax.experimental.pallas import tpu_sc as plsc

_N, _E, _DN, _DE, _H, _NG = 10000, 320000, 128, 16, 128, 16

_NBLK = 1000
_EBLK = 1280
_GCH = 64
_NGCHUNK = _E // _GCH
_NW = 32
_NPAD = 10240
_RPT = _NPAD // 16


def _rne_bf16_bits(x):
    b = lax.bitcast_convert_type(x, jnp.int32)
    return b + 0x7FFF + (lax.shift_right_logical(b, 16) & 1)


def _pack2bf16(f):
    h = f.shape[1] // 2
    lo = lax.shift_right_logical(_rne_bf16_bits(f[:, :h]), 16)
    hi = _rne_bf16_bits(f[:, h:]) & jnp.int32(-65536)
    return lo | hi


def _unpack2bf16(w):
    lo = lax.bitcast_convert_type(lax.shift_left(w, 16), jnp.float32)
    hi = lax.bitcast_convert_type(w & jnp.int32(-65536), jnp.float32)
    return lo, hi


def _ln_rows(y, g, b):
    m = jnp.mean(y, axis=-1, keepdims=True)
    v = jnp.mean((y - m) ** 2, axis=-1, keepdims=True)
    return (y - m) / jnp.sqrt(v + 1e-5) * g + b


def _dot16(a, b):
    return jnp.dot(a.astype(jnp.bfloat16), b.astype(jnp.bfloat16),
                   preferred_element_type=jnp.float32)


def _mlp_ln_body(x_ref, w1, b1, w2, b2, g, be, o_ref):
    h = jnp.maximum(_dot16(x_ref[...], w1[...]) + b1[...], 0.0)
    y = _dot16(h, w2[...]) + b2[...]
    o_ref[...] = _ln_rows(y, g[...], be[...])


def _encoder(x, w1t, b1, w2t, b2, g, be, blk, grid):
    din = x.shape[1]
    return pl.pallas_call(
        _mlp_ln_body,
        grid=(grid,),
        in_specs=[
            pl.BlockSpec((blk, din), lambda i: (i, 0)),
            pl.BlockSpec((din, _H), lambda i: (0, 0)),
            pl.BlockSpec((1, _H), lambda i: (0, 0)),
            pl.BlockSpec((_H, _H), lambda i: (0, 0)),
            pl.BlockSpec((1, _H), lambda i: (0, 0)),
            pl.BlockSpec((1, _H), lambda i: (0, 0)),
            pl.BlockSpec((1, _H), lambda i: (0, 0)),
        ],
        out_specs=pl.BlockSpec((blk, _H), lambda i: (i, 0)),
        out_shape=jax.ShapeDtypeStruct((x.shape[0], _H), jnp.float32),
    )(x, w1t, b1, w2t, b2, g, be)


def _node_mlp_body(eps_ref, x_ref, a_ref, w1, b1, w2, b2, g, be, o_ref,
                   *, post_relu):
    z = eps_ref[0, 0] * x_ref[...] + a_ref[0] + a_ref[1]
    h = jnp.maximum(_dot16(z, w1[...]) + b1[...], 0.0)
    y = _dot16(h, w2[...]) + b2[...]
    y = _ln_rows(y, g[...], be[...])
    if post_relu:
        y = jnp.maximum(y, 0.0)
    o_ref[...] = y


def _node_mlp(eps1, x, aggr2, w1t, b1, w2t, b2, g, be, post_relu):
    return pl.pallas_call(
        functools.partial(_node_mlp_body, post_relu=post_relu),
        grid=(_N // _NBLK,),
        in_specs=[
            pl.BlockSpec(memory_space=pltpu.SMEM),
            pl.BlockSpec((_NBLK, _H), lambda i: (i, 0)),
            pl.BlockSpec((2, _NBLK, _H), lambda i: (0, i, 0)),
            pl.BlockSpec((_H, _H), lambda i: (0, 0)),
            pl.BlockSpec((1, _H), lambda i: (0, 0)),
            pl.BlockSpec((_H, _H), lambda i: (0, 0)),
            pl.BlockSpec((1, _H), lambda i: (0, 0)),
            pl.BlockSpec((1, _H), lambda i: (0, 0)),
            pl.BlockSpec((1, _H), lambda i: (0, 0)),
        ],
        out_specs=pl.BlockSpec((_NBLK, _H), lambda i: (i, 0)),
        out_shape=jax.ShapeDtypeStruct((_N, _H), jnp.float32),
    )(eps1, x, aggr2, w1t, b1, w2t, b2, g, be)


def _pool_sums_body(x_ref, b_ref, sums_ref, cnt_ref):
    i = pl.program_id(0)
    bb = b_ref[...]
    iota = lax.broadcasted_iota(jnp.int32, (_NBLK, _NG), 1)
    onehot = (bb == iota).astype(jnp.float32)
    dn = (((0,), (0,)), ((), ()))
    s = lax.dot_general(onehot, x_ref[...], dn,
                        preferred_element_type=jnp.float32)
    c = lax.dot_general(onehot, jnp.ones_like(x_ref[...]), dn,
                        preferred_element_type=jnp.float32)

    @pl.when(i == 0)
    def _():
        sums_ref[...] = s
        cnt_ref[...] = c

    @pl.when(i > 0)
    def _():
        sums_ref[...] += s
        cnt_ref[...] += c


def _pool_sums(x2, batch2d):
    return pl.pallas_call(
        _pool_sums_body,
        grid=(_N // _NBLK,),
        in_specs=[
            pl.BlockSpec((_NBLK, _H), lambda i: (i, 0)),
            pl.BlockSpec((_NBLK, 1), lambda i: (i, 0)),
        ],
        out_specs=[
            pl.BlockSpec((_NG, _H), lambda i: (0, 0)),
            pl.BlockSpec((_NG, _H), lambda i: (0, 0)),
        ],
        out_shape=[
            jax.ShapeDtypeStruct((_NG, _H), jnp.float32),
            jax.ShapeDtypeStruct((_NG, _H), jnp.float32),
        ],
    )(x2, batch2d)


def _graph_mlp_body(sums_ref, cnt_ref, wt, b, g, be, wct, gc_ref):
    gmean = sums_ref[...] / jnp.maximum(cnt_ref[...], 1.0)
    t = jnp.maximum(
        jnp.dot(gmean, wt[...], preferred_element_type=jnp.float32)
        + b[...], 0.0)
    gg = _ln_rows(t, g[...], be[...])
    gc_ref[...] = jnp.dot(gg, wct[...], preferred_element_type=jnp.float32)


def _graph_mlp(sums, cnt, wt, b, g, be, wct):
    return pl.pallas_call(
        _graph_mlp_body,
        out_shape=jax.ShapeDtypeStruct((_NG, 2 * _H), jnp.float32),
    )(sums, cnt, wt, b, g, be, wct)


def _proj_body(x_ref, b_ref, gc_ref, wat, wbt, p_ref, q_ref):
    bb = b_ref[...]
    iota = lax.broadcasted_iota(jnp.int32, (_NBLK, _NG), 1)
    onehot = (bb == iota).astype(jnp.float32)
    xb = x_ref[...]
    pf = (_dot16(xb, wat[...])
          + jnp.dot(onehot, gc_ref[...], preferred_element_type=jnp.float32))
    qf = _dot16(xb, wbt[...])
    p_ref[...] = _pack2bf16(pf)
    q_ref[...] = _pack2bf16(qf)


def _projections(x2, batch2d, gc, wat, wbt):
    return pl.pallas_call(
        _proj_body,
        grid=(_N // _NBLK,),
        in_specs=[
            pl.BlockSpec((_NBLK, _H), lambda i: (i, 0)),
            pl.BlockSpec((_NBLK, 1), lambda i: (i, 0)),
            pl.BlockSpec((_NG, 2 * _H), lambda i: (0, 0)),
            pl.BlockSpec((_H, 2 * _H), lambda i: (0, 0)),
            pl.BlockSpec((_H, 2 * _H), lambda i: (0, 0)),
        ],
        out_specs=[
            pl.BlockSpec((_NBLK, _H), lambda i: (i, 0)),
            pl.BlockSpec((_NBLK, _H), lambda i: (i, 0)),
        ],
        out_shape=[
            jax.ShapeDtypeStruct((_N, _H), jnp.int32),
            jax.ShapeDtypeStruct((_N, _H), jnp.int32),
        ],
    )(x2, batch2d, gc, wat, wbt)


def _final_body(e_ref, s_ref, wdlo, wdhi, b1lo, b1hi, w2lo, w2hi,
                b2, w3, b3, o_ref):
    e16 = e_ref[...].astype(jnp.bfloat16)
    telo = jnp.dot(e16, wdlo[...], preferred_element_type=jnp.float32)
    tehi = jnp.dot(e16, wdhi[...], preferred_element_type=jnp.float32)
    slo, shi = _unpack2bf16(s_ref[...])
    h1lo = jnp.tanh(slo + telo + b1lo[...])
    h1hi = jnp.tanh(shi + tehi + b1hi[...])
    h2 = jnp.tanh(
        jnp.dot(h1lo.astype(jnp.bfloat16), w2lo[...],
                preferred_element_type=jnp.float32)
        + jnp.dot(h1hi.astype(jnp.bfloat16), w2hi[...],
                  preferred_element_type=jnp.float32)
        + b2[...])
    o_ref[...] = jax.nn.sigmoid(
        jnp.sum(h2 * w3[...], axis=1, keepdims=True) + b3[...])


def _final_mlp(e_emb, s3, base, ne, *ws):
    wspec = pl.BlockSpec((_H, _H), lambda i: (0, 0))
    bspec = pl.BlockSpec((1, _H), lambda i: (0, 0))
    bblk = base // _EBLK
    return pl.pallas_call(
        _final_body,
        grid=(ne // _EBLK,),
        in_specs=[
            pl.BlockSpec((_EBLK, _H), lambda i: (i + bblk, 0)),
            pl.BlockSpec((_EBLK, _H), lambda i: (i, 0)),
            wspec, wspec, bspec, bspec, wspec, wspec, bspec, bspec,
            pl.BlockSpec((1, 1), lambda i: (0, 0)),
        ],
        out_specs=pl.BlockSpec((_EBLK, 1), lambda i: (i, 0)),
        out_shape=jax.ShapeDtypeStruct((ne, 1), jnp.float32),
    )(e_emb, s3, *ws)


def _sc_mesh():
    return plsc.VectorSubcoreMesh(core_axis_name="c", subcore_axis_name="s")


def _sc_no_layout_params():
    cp = pltpu.CompilerParams()
    if "needs_layout_passes" in pltpu.CompilerParams.__dataclass_fields__:
        cp = dataclasses.replace(cp, needs_layout_passes=False)
    return cp


def _gine_aggregate(x_emb, e_emb, src, dst):
    K = _NGCHUNK // _NW
    NREM = _NGCHUNK - K * _NW

    @functools.partial(
        pl.kernel,
        out_type=jax.ShapeDtypeStruct((2, _NPAD, _H), jnp.float32),
        mesh=_sc_mesh(),
        scratch_types=[
            pltpu.VMEM((_GCH,), jnp.int32),
            pltpu.VMEM((_GCH,), jnp.int32),
            pltpu.VMEM((_GCH,), jnp.int32),
            pltpu.VMEM((_GCH,), jnp.int32),
            pltpu.VMEM((_GCH, _H), jnp.float32),
            pltpu.VMEM((_GCH, _H), jnp.float32),
            pltpu.VMEM((_GCH, _H), jnp.float32),
            pltpu.VMEM((_GCH, _H), jnp.float32),
            pltpu.VMEM_SHARED((_NPAD, _H), jnp.float32),
            pltpu.SemaphoreType.DMA,
            pltpu.SemaphoreType.DMA,
            pltpu.SemaphoreType.DMA,
            pltpu.SemaphoreType.DMA,
            pltpu.SemaphoreType.DMA,
            pltpu.SemaphoreType.DMA,
            pltpu.SemaphoreType.DMA,
        ],
    )
    def k(x_hbm, e_hbm, src_hbm, dst_hbm, out_hbm,
          sidx0, sidx1, didx0, didx1, xr0, xr1, er0, er1, accum,
          semi0, semi1, semg0, semg1, seme0, seme1, semw):
        semi, semg, seme = [semi0, semi1], [semg0, semg1], [seme0, seme1]
        sidx, didx, xr, er = [sidx0, sidx1], [didx0, didx1], [xr0, xr1], [er0, er1]
        cid = lax.axis_index("c")
        sid = lax.axis_index("s")
        wid = sid * 2 + cid

        @pl.loop(0, _GCH)
        def _(r):
            for c in range(_H // 16):
                er0[r, pl.ds(c * 16, 16)] = jnp.zeros((16,), jnp.float32)

        for j in range(_RPT // _GCH):
            pltpu.sync_copy(er0, accum.at[pl.ds(sid * _RPT + j * _GCH, _GCH)])
        plsc.subcore_barrier()

        def off(j):
            return (j * _NW + wid) * _GCH

        def issue_idx(j, b):
            pltpu.async_copy(src_hbm.at[pl.ds(off(j), _GCH)], sidx[b], semi[b])
            pltpu.async_copy(dst_hbm.at[pl.ds(off(j), _GCH)], didx[b], semi[b])

        def wait_idx(j, b):
            pltpu.make_async_copy(src_hbm.at[pl.ds(off(j), _GCH)], sidx[b],
                                  semi[b]).wait()
            pltpu.make_async_copy(dst_hbm.at[pl.ds(off(j), _GCH)], didx[b],
                                  semi[b]).wait()

        def issue_gather(j, b):
            pltpu.async_copy(x_hbm.at[sidx[b]], xr[b], semg[b])
            pltpu.async_copy(e_hbm.at[pl.ds(off(j), _GCH)], er[b], seme[b])

        def wait_gather(j, b):
            pltpu.make_async_copy(x_hbm.at[sidx[b]], xr[b], semg[b]).wait()
            pltpu.make_async_copy(e_hbm.at[pl.ds(off(j), _GCH)], er[b],
                                  seme[b]).wait()

        def compute(b):
            @pl.loop(0, _GCH)
            def _(r):
                for c in range(_H // 16):
                    sl = pl.ds(c * 16, 16)
                    xr[b][r, sl] = jnp.maximum(xr[b][r, sl] + er[b][r, sl],
                                               0.0)

        issue_idx(0, 0)
        issue_idx(1, 1)
        wait_idx(0, 0)
        issue_gather(0, 0)

        @pl.loop(0, K // 2)
        def _(jj):
            for u in range(2):
                j = 2 * jj + u
                b, nb = u, 1 - u

                @pl.when(j + 1 < K)
                def _():
                    wait_idx(j + 1, nb)
                    issue_gather(j + 1, nb)

                wait_gather(j, b)
                compute(b)
                pltpu.async_copy(xr[b], accum.at[didx[b]], semw, add=True)
                pltpu.make_async_copy(xr[b], accum.at[didx[b]], semw).wait()

                @pl.when(j + 2 < K)
                def _():
                    issue_idx(j + 2, b)

        @pl.when(wid < NREM)
        def _():
            o = (K * _NW + wid) * _GCH
            pltpu.sync_copy(src_hbm.at[pl.ds(o, _GCH)], sidx0)
            pltpu.sync_copy(dst_hbm.at[pl.ds(o, _GCH)], didx0)
            pltpu.async_copy(x_hbm.at[sidx0], xr0, semg0).wait()
            pltpu.sync_copy(e_hbm.at[pl.ds(o, _GCH)], er0)

            @pl.loop(0, _GCH)
            def _(r):
                for c in range(_H // 16):
                    sl = pl.ds(c * 16, 16)
                    xr0[r, sl] = jnp.maximum(xr0[r, sl] + er0[r, sl], 0.0)

            pltpu.sync_copy(xr0, accum.at[didx0], add=True)

        plsc.subcore_barrier()
        pltpu.sync_copy(accum.at[pl.ds(sid * _RPT, _RPT)],
                        out_hbm.at[cid, pl.ds(sid * _RPT, _RPT)])

    return k(x_emb, e_emb, src, dst)


_CCH = 128
_NCCHUNK = _E // _CCH


def _edge_combine(p, q, src, dst, ne):
    nchunk = ne // _CCH
    K = (nchunk // _NW) & ~1
    NREM = nchunk - K * _NW

    @functools.partial(
        pl.kernel,
        out_type=jax.ShapeDtypeStruct((ne, _H), jnp.int32),
        mesh=_sc_mesh(),
        compiler_params=_sc_no_layout_params(),
        scratch_types=[
            pltpu.VMEM((_CCH,), jnp.int32),
            pltpu.VMEM((_CCH,), jnp.int32),
            pltpu.VMEM((_CCH,), jnp.int32),
            pltpu.VMEM((_CCH,), jnp.int32),
            pltpu.VMEM((_CCH, _H), jnp.int32),
            pltpu.VMEM((_CCH, _H), jnp.int32),
            pltpu.VMEM((_CCH, _H), jnp.int32),
            pltpu.VMEM((_CCH, _H), jnp.int32),
            pltpu.SemaphoreType.DMA,
            pltpu.SemaphoreType.DMA,
            pltpu.SemaphoreType.DMA,
            pltpu.SemaphoreType.DMA,
            pltpu.SemaphoreType.DMA,
        ],
    )
    def k(p_hbm, q_hbm, src_hbm, dst_hbm, out_hbm,
          sidx0, sidx1, didx0, didx1, pr0, pr1, qr0, qr1,
          semi0, semi1, semg0, semg1, semw):
        semi, semg = [semi0, semi1], [semg0, semg1]
        sidx, didx, pr, qr = [sidx0, sidx1], [didx0, didx1], [pr0, pr1], [qr0, qr1]
        cid = lax.axis_index("c")
        sid = lax.axis_index("s")
        wid = sid * 2 + cid

        def off(j):
            return (j * _NW + wid) * _CCH

        def issue_idx(j, b):
            pltpu.async_copy(src_hbm.at[pl.ds(off(j), _CCH)], sidx[b], semi[b])
            pltpu.async_copy(dst_hbm.at[pl.ds(off(j), _CCH)], didx[b], semi[b])

        def wait_idx(j, b):
            pltpu.make_async_copy(src_hbm.at[pl.ds(off(j), _CCH)], sidx[b],
                                  semi[b]).wait()
            pltpu.make_async_copy(dst_hbm.at[pl.ds(off(j), _CCH)], didx[b],
                                  semi[b]).wait()

        def issue_gather(b):
            pltpu.async_copy(p_hbm.at[sidx[b]], pr[b], semg[b])
            pltpu.async_copy(q_hbm.at[didx[b]], qr[b], semg[b])

        def wait_gather(b):
            pltpu.make_async_copy(p_hbm.at[sidx[b]], pr[b], semg[b]).wait()
            pltpu.make_async_copy(q_hbm.at[didx[b]], qr[b], semg[b]).wait()

        def compute(b):
            @pl.loop(0, _CCH)
            def _(r):
                for c in range(_H // 16):
                    sl = pl.ds(c * 16, 16)
                    s = (plsc.bitcast(pr[b][r, sl], jnp.bfloat16)
                         + plsc.bitcast(qr[b][r, sl], jnp.bfloat16))
                    pr[b][r, sl] = plsc.bitcast(s, jnp.int32)

        issue_idx(0, 0)
        issue_idx(1, 1)
        wait_idx(0, 0)
        issue_gather(0)

        @pl.loop(0, K // 2)
        def _(jj):
            for u in range(2):
                j = 2 * jj + u
                b, nb = u, 1 - u

                @pl.when(j + 1 < K)
                def _():
                    wait_idx(j + 1, nb)
                    issue_gather(nb)

                wait_gather(b)
                compute(b)
                pltpu.async_copy(pr[b], out_hbm.at[pl.ds(off(j), _CCH)],
                                 semw)
                pltpu.make_async_copy(pr[b],
                                      out_hbm.at[pl.ds(off(j), _CCH)],
                                      semw).wait()

                @pl.when(j + 2 < K)
                def _():
                    issue_idx(j + 2, b)

        for t in range((NREM + _NW - 1) // _NW):
            @pl.when((K + t) * _NW + wid < nchunk)
            def _():
                o = ((K + t) * _NW + wid) * _CCH
                pltpu.sync_copy(src_hbm.at[pl.ds(o, _CCH)], sidx0)
                pltpu.sync_copy(dst_hbm.at[pl.ds(o, _CCH)], didx0)
                pltpu.async_copy(p_hbm.at[sidx0], pr0, semg0).wait()
                pltpu.async_copy(q_hbm.at[didx0], qr0, semg0).wait()

                @pl.loop(0, _CCH)
                def _(r):
                    for c in range(_H // 16):
                        sl = pl.ds(c * 16, 16)
                        s = (plsc.bitcast(pr0[r, sl], jnp.bfloat16)
                             + plsc.bitcast(qr0[r, sl], jnp.bfloat16))
                        pr0[r, sl] = plsc.bitcast(s, jnp.int32)

                pltpu.sync_copy(pr0, out_hbm.at[pl.ds(o, _CCH)])

    return k(p, q, src, dst)


def kernel(x, edge_index, edge_attr, batch, params):
    p = params
    src = edge_index[0]
    dst = edge_index[1]
    batch2d = batch.astype(jnp.int32).reshape(_N, 1)

    def row(v):
        return v.reshape(1, -1)

    x_emb = _encoder(x, p['ne_w1'].T, row(p['ne_b1']), p['ne_w2'].T,
                     row(p['ne_b2']), row(p['ne_g']), row(p['ne_be']),
                     _NBLK, _N // _NBLK)
    e_emb = _encoder(edge_attr, p['ee_w1'].T, row(p['ee_b1']), p['ee_w2'].T,
                     row(p['ee_b2']), row(p['ee_g']), row(p['ee_be']),
                     _EBLK, _E // _EBLK)

    x_curr = x_emb
    for i in range(2):
        aggr2 = _gine_aggregate(x_curr, e_emb, src, dst)
        eps1 = (1.0 + p['gin%d_eps' % i]).astype(jnp.float32).reshape(1, 1)
        x_curr = _node_mlp(
            eps1, x_curr, aggr2,
            p['gin%d_w1' % i].T, row(p['gin%d_b1' % i]),
            p['gin%d_w2' % i].T, row(p['gin%d_b2' % i]),
            row(p['gin%d_g' % i]), row(p['gin%d_be' % i]),
            post_relu=(i < 1))

    sums, cnt = _pool_sums(x_curr, batch2d)
    wc_t = p['ep_w1'][:, 2 * _H:3 * _H].T
    gc = _graph_mlp(sums, cnt, p['gp_w'].T, row(p['gp_b']), row(p['gp_g']),
                    row(p['gp_be']), wc_t)

    wa_t = p['ep_w1'][:, :_H].T
    wb_t = p['ep_w1'][:, _H:2 * _H].T
    pn, qn = _projections(x_curr, batch2d, gc, wa_t, wb_t)

    wd_t = p['ep_w1'][:, 3 * _H:].T.astype(jnp.bfloat16)
    w2_t = p['ep_w2'].T.astype(jnp.bfloat16)
    ws = (wd_t[:, :_H], wd_t[:, _H:],
          row(p['ep_b1'][:_H]), row(p['ep_b1'][_H:]),
          w2_t[:_H, :], w2_t[_H:, :],
          row(p['ep_b2']), row(p['ep_w3']),
          p['ep_b3'].reshape(1, 1))
    ea = 81920
    ranges = [(i * ea, ea) for i in range(3)] + [(3 * ea, _E - 3 * ea)]
    outs = []
    for base, ne in ranges:
        s3 = _edge_combine(pn, qn, src[base:base + ne], dst[base:base + ne],
                           ne)
        outs.append(_final_mlp(e_emb, s3, base, ne, *ws))
    return jnp.concatenate(outs, axis=0)
